# baseline (speedup 1.0000x reference)
import functools

import jax
import jax.numpy as jnp
from jax import lax
from jax.experimental import pallas as pl
from jax.experimental.pallas import tpu as pltpu
from jax.experimental.pallas import tpu_sc as plsc

D = 64
DP = 128
BB = 128
NC = 2
NS = 16
NW = NC * NS


@functools.cache
def _gather_call(b, s):
    blk_w = (b // BB) // NW
    per_s = blk_w * BB
    assert blk_w * BB * NW == b and blk_w == 4 and s % 2 == 0
    rounds = s // 2
    mesh = plsc.VectorSubcoreMesh(core_axis_name="c", subcore_axis_name="s")

    @functools.partial(
        pl.kernel,
        mesh=mesh,
        out_type=jax.ShapeDtypeStruct((s, D, b), jnp.float32),
        scratch_types=[
            pltpu.VMEM((per_s,), jnp.int32),
            pltpu.VMEM((per_s,), jnp.int32),
            pltpu.VMEM((BB, DP + 1), jnp.float32),
            pltpu.VMEM((BB, DP + 1), jnp.float32),
            pltpu.VMEM((2, D, BB), jnp.float32),
        ]
        + [pltpu.SemaphoreType.DMA] * 6,
        compiler_params=pltpu.CompilerParams(needs_layout_passes=False),
    )
    def k(table_hbm, idx_hbm, out_hbm, idx_v0, idx_v1, rows_f0, rows_f1,
          tr_v, *sems):
        idx_vs = (idx_v0, idx_v1)
        rows_views = (rows_f0, rows_f1)
        gsem = sems[0:2]
        osem = sems[2:4]
        isem = sems[4:6]
        wid = lax.axis_index("s") * NC + lax.axis_index("c")
        b0 = wid * per_s

        lane = lax.iota(jnp.int32, 16)
        rowv = [lane + 16 * j for j in range(BB // 16)]

        def stage_idx(sq, islot):
            pltpu.async_copy(
                idx_hbm.at[pl.ds(sq * b + b0, per_s)],
                idx_vs[islot], isem[islot])

        def wait_idx(islot):
            pltpu.make_async_copy(
                idx_hbm.at[pl.ds(0, per_s)], idx_vs[islot],
                isem[islot]).wait()

        def fire(t, slot, sbase):
            islot = (t // 4) & 1
            off = (t % 4) * BB
            pltpu.async_copy(
                table_hbm.at[idx_vs[islot].at[pl.ds(off, BB)]],
                rows_views[slot].at[:, pl.ds(0, DP)], gsem[slot])

        def drain_gather(slot):
            pltpu.make_async_copy(
                table_hbm.at[pl.ds(0, BB)],
                rows_views[slot].at[:, pl.ds(0, DP)], gsem[slot]).wait()

        def transpose(slot):
            src = rows_views[slot]

            def tbody(d, carry):
                col = jnp.zeros((16,), jnp.int32) + d
                for j in range(BB // 16):
                    v = plsc.load_gather(src, [rowv[j], col])
                    tr_v[slot, d, pl.ds(16 * j, 16)] = v
                return carry

            lax.fori_loop(0, D, tbody, 0)

        def start_out(sq, bb, slot):
            pltpu.async_copy(
                tr_v.at[slot],
                out_hbm.at[sq, :, pl.ds(b0 + bb * BB, BB)],
                osem[slot])

        def wait_out(slot):
            pltpu.make_async_copy(
                tr_v.at[slot], out_hbm.at[0, :, pl.ds(0, BB)],
                osem[slot]).wait()

        def round_body(r, first=False, last=False):
            for t in range(8):
                slot = t & 1
                sq = 2 * r + t // 4
                drain_gather(slot)
                if t == 3 and not last:
                    stage_idx(sq + 2, 0)
                if t == 7 and not last:
                    stage_idx(sq + 2, 1)
                if not (first and t < 2):
                    wait_out(slot)
                transpose(slot)
                start_out(sq, t % 4, slot)
                if t == 2:
                    wait_idx(1)
                if t == 6 and not last:
                    wait_idx(0)
                if not (last and t >= 6):
                    fire((t + 2) % 8, slot, 2 * r)

        pltpu.sync_copy(idx_hbm.at[pl.ds(b0, per_s)], idx_vs[0])
        stage_idx(1, 1)
        fire(0, 0, 0)
        fire(1, 1, 0)

        round_body(0, first=True)

        def loop_body(r, carry):
            round_body(r)
            return carry

        lax.fori_loop(1, rounds - 1, loop_body, 0)

        round_body(rounds - 1, last=True)
        wait_out(0)
        wait_out(1)

    return k


@jax.jit
def kernel(pos_ids, position_encoding):
    b, s = pos_ids.shape
    idx = pos_ids.T.reshape(s * b).astype(jnp.int32)
    table = jnp.pad(position_encoding.astype(jnp.float32),
                    ((0, 0), (0, DP - D)))
    out_t = _gather_call(b, s)(table, idx)
    return out_t.transpose(2, 0, 1)


# speedup vs baseline: 1.2257x; 1.2257x over previous
import functools

import jax
import jax.numpy as jnp
from jax import lax
from jax.experimental import pallas as pl
from jax.experimental.pallas import tpu as pltpu
from jax.experimental.pallas import tpu_sc as plsc

D = 64
DP = 128
BB = 128
NC = 2
NS = 16
NW = NC * NS


@functools.cache
def _gather_call(b, s):
    blk_w = (b // BB) // NW
    per_s = blk_w * BB
    assert blk_w * BB * NW == b and blk_w == 4 and s % 2 == 0
    rounds = s // 2
    mesh = plsc.VectorSubcoreMesh(core_axis_name="c", subcore_axis_name="s")

    @functools.partial(
        pl.kernel,
        mesh=mesh,
        out_type=jax.ShapeDtypeStruct((s, D, b), jnp.float32),
        scratch_types=[
            pltpu.VMEM((per_s,), jnp.int32),
            pltpu.VMEM((per_s,), jnp.int32),
            pltpu.VMEM((BB, DP + 1), jnp.float32),
            pltpu.VMEM((BB, DP + 1), jnp.float32),
            pltpu.VMEM((2, D, BB), jnp.float32),
        ]
        + [pltpu.SemaphoreType.DMA] * 6,
        compiler_params=pltpu.CompilerParams(needs_layout_passes=False),
    )
    def k(table_hbm, idx_hbm, out_hbm, idx_v0, idx_v1, rows_f0, rows_f1,
          tr_v, *sems):
        idx_vs = (idx_v0, idx_v1)
        rows_views = (rows_f0, rows_f1)
        gsem = sems[0:2]
        osem = sems[2:4]
        isem = sems[4:6]
        wid = lax.axis_index("s") * NC + lax.axis_index("c")
        b0 = wid * per_s

        lane = lax.iota(jnp.int32, 16)
        colv = [lane + 16 * c for c in range(D // 16)]

        def stage_idx(sq, islot):
            pltpu.async_copy(
                idx_hbm.at[pl.ds(sq * b + b0, per_s)],
                idx_vs[islot], isem[islot])

        def wait_idx(islot):
            pltpu.make_async_copy(
                idx_hbm.at[pl.ds(0, per_s)], idx_vs[islot],
                isem[islot]).wait()

        def fire(t, slot, sbase):
            islot = (t // 4) & 1
            off = (t % 4) * BB
            pltpu.async_copy(
                table_hbm.at[idx_vs[islot].at[pl.ds(off, BB)]],
                rows_views[slot].at[:, pl.ds(0, DP)], gsem[slot])

        def drain_gather(slot):
            pltpu.make_async_copy(
                table_hbm.at[pl.ds(0, BB)],
                rows_views[slot].at[:, pl.ds(0, DP)], gsem[slot]).wait()

        def transpose(slot):
            src = rows_views[slot]
            dst = tr_v.at[slot]

            def tbody(r, carry):
                row = jnp.zeros((16,), jnp.int32) + r
                for c in range(D // 16):
                    v = src[r, pl.ds(16 * c, 16)]
                    plsc.store_scatter(dst, [colv[c], row], v)
                return carry

            lax.fori_loop(0, BB, tbody, 0)

        def start_out(sq, bb, slot):
            pltpu.async_copy(
                tr_v.at[slot],
                out_hbm.at[sq, :, pl.ds(b0 + bb * BB, BB)],
                osem[slot])

        def wait_out(slot):
            pltpu.make_async_copy(
                tr_v.at[slot], out_hbm.at[0, :, pl.ds(0, BB)],
                osem[slot]).wait()

        def round_body(r, first=False, last=False):
            for t in range(8):
                slot = t & 1
                sq = 2 * r + t // 4
                drain_gather(slot)
                if t == 3 and not last:
                    stage_idx(sq + 2, 0)
                if t == 7 and not last:
                    stage_idx(sq + 2, 1)
                if not (first and t < 2):
                    wait_out(slot)
                transpose(slot)
                start_out(sq, t % 4, slot)
                if t == 2:
                    wait_idx(1)
                if t == 6 and not last:
                    wait_idx(0)
                if not (last and t >= 6):
                    fire((t + 2) % 8, slot, 2 * r)

        pltpu.sync_copy(idx_hbm.at[pl.ds(b0, per_s)], idx_vs[0])
        stage_idx(1, 1)
        fire(0, 0, 0)
        fire(1, 1, 0)

        round_body(0, first=True)

        def loop_body(r, carry):
            round_body(r)
            return carry

        lax.fori_loop(1, rounds - 1, loop_body, 0)

        round_body(rounds - 1, last=True)
        wait_out(0)
        wait_out(1)

    return k


@jax.jit
def kernel(pos_ids, position_encoding):
    b, s = pos_ids.shape
    idx = pos_ids.T.reshape(s * b).astype(jnp.int32)
    table = jnp.pad(position_encoding.astype(jnp.float32),
                    ((0, 0), (0, DP - D)))
    out_t = _gather_call(b, s)(table, idx)
    return out_t.transpose(2, 0, 1)


# speedup vs baseline: 1.2265x; 1.0007x over previous
import functools

import jax
import jax.numpy as jnp
from jax import lax
from jax.experimental import pallas as pl
from jax.experimental.pallas import tpu as pltpu
from jax.experimental.pallas import tpu_sc as plsc

D = 64
DP = 128
BB = 128
NC = 2
NS = 16
NW = NC * NS


@functools.cache
def _gather_call(b, s):
    blk_w = (b // BB) // NW
    per_s = blk_w * BB
    assert blk_w * BB * NW == b and blk_w == 4 and s % 2 == 0
    rounds = s // 2
    mesh = plsc.VectorSubcoreMesh(core_axis_name="c", subcore_axis_name="s")

    @functools.partial(
        pl.kernel,
        mesh=mesh,
        out_type=jax.ShapeDtypeStruct((s, D, b), jnp.float32),
        scratch_types=[
            pltpu.VMEM((per_s,), jnp.int32),
            pltpu.VMEM((per_s,), jnp.int32),
            pltpu.VMEM((BB, DP + 1), jnp.float32),
            pltpu.VMEM((BB, DP + 1), jnp.float32),
            pltpu.VMEM((2, D, BB + 1), jnp.float32),
        ]
        + [pltpu.SemaphoreType.DMA] * 6,
        compiler_params=pltpu.CompilerParams(needs_layout_passes=False),
    )
    def k(table_hbm, idx_hbm, out_hbm, idx_v0, idx_v1, rows_f0, rows_f1,
          tr_v, *sems):
        idx_vs = (idx_v0, idx_v1)
        rows_views = (rows_f0, rows_f1)
        gsem = sems[0:2]
        osem = sems[2:4]
        isem = sems[4:6]
        wid = lax.axis_index("s") * NC + lax.axis_index("c")
        b0 = wid * per_s

        lane = lax.iota(jnp.int32, 16)
        colv = [lane + 16 * c for c in range(D // 16)]

        def stage_idx(sq, islot):
            pltpu.async_copy(
                idx_hbm.at[pl.ds(sq * b + b0, per_s)],
                idx_vs[islot], isem[islot])

        def wait_idx(islot):
            pltpu.make_async_copy(
                idx_hbm.at[pl.ds(0, per_s)], idx_vs[islot],
                isem[islot]).wait()

        def fire(t, slot, sbase):
            islot = (t // 4) & 1
            off = (t % 4) * BB
            pltpu.async_copy(
                table_hbm.at[idx_vs[islot].at[pl.ds(off, BB)]],
                rows_views[slot].at[:, pl.ds(0, DP)], gsem[slot])

        def drain_gather(slot):
            pltpu.make_async_copy(
                table_hbm.at[pl.ds(0, BB)],
                rows_views[slot].at[:, pl.ds(0, DP)], gsem[slot]).wait()

        def transpose(slot):
            src = rows_views[slot]
            dst = tr_v.at[slot]

            def tbody(r, carry):
                row = jnp.zeros((16,), jnp.int32) + r
                for c in range(D // 16):
                    v = src[r, pl.ds(16 * c, 16)]
                    plsc.store_scatter(dst, [colv[c], row], v)
                return carry

            lax.fori_loop(0, BB, tbody, 0)

        def start_out(sq, bb, slot):
            pltpu.async_copy(
                tr_v.at[slot, :, pl.ds(0, BB)],
                out_hbm.at[sq, :, pl.ds(b0 + bb * BB, BB)],
                osem[slot])

        def wait_out(slot):
            pltpu.make_async_copy(
                tr_v.at[slot, :, pl.ds(0, BB)],
                out_hbm.at[0, :, pl.ds(0, BB)],
                osem[slot]).wait()

        def round_body(r, first=False, last=False):
            for t in range(8):
                slot = t & 1
                sq = 2 * r + t // 4
                drain_gather(slot)
                if t == 3 and not last:
                    stage_idx(sq + 2, 0)
                if t == 7 and not last:
                    stage_idx(sq + 2, 1)
                if not (first and t < 2):
                    wait_out(slot)
                transpose(slot)
                start_out(sq, t % 4, slot)
                if t == 2:
                    wait_idx(1)
                if t == 6 and not last:
                    wait_idx(0)
                if not (last and t >= 6):
                    fire((t + 2) % 8, slot, 2 * r)

        pltpu.sync_copy(idx_hbm.at[pl.ds(b0, per_s)], idx_vs[0])
        stage_idx(1, 1)
        fire(0, 0, 0)
        fire(1, 1, 0)

        round_body(0, first=True)

        def loop_body(r, carry):
            round_body(r)
            return carry

        lax.fori_loop(1, rounds - 1, loop_body, 0)

        round_body(rounds - 1, last=True)
        wait_out(0)
        wait_out(1)

    return k


@jax.jit
def kernel(pos_ids, position_encoding):
    b, s = pos_ids.shape
    idx = pos_ids.T.reshape(s * b).astype(jnp.int32)
    table = jnp.pad(position_encoding.astype(jnp.float32),
                    ((0, 0), (0, DP - D)))
    out_t = _gather_call(b, s)(table, idx)
    return out_t.transpose(2, 0, 1)


# speedup vs baseline: 2.1042x; 1.7156x over previous
import functools

import jax
import jax.numpy as jnp
from jax import lax
from jax.experimental import pallas as pl
from jax.experimental.pallas import tpu as pltpu
from jax.experimental.pallas import tpu_sc as plsc

D = 64
DP = 128
S = 200
GB = 16
NC = 2
NS = 16
NW = NC * NS


@functools.cache
def _gather_call(b):
    per_w = b // NW
    n_groups = per_w // GB
    assert per_w * NW == b and n_groups * GB == per_w
    assert n_groups % 2 == 0 and n_groups >= 6
    mesh = plsc.VectorSubcoreMesh(core_axis_name="c", subcore_axis_name="s")

    @functools.partial(
        pl.kernel,
        mesh=mesh,
        out_type=jax.ShapeDtypeStruct((b, S, D), jnp.float32),
        scratch_types=[
            pltpu.VMEM((GB * S,), jnp.int32),
            pltpu.VMEM((GB * S,), jnp.int32),
            pltpu.VMEM((2, S, DP), jnp.float32),
            pltpu.VMEM((2, S, D), jnp.float32),
        ]
        + [pltpu.SemaphoreType.DMA] * 6,
    )
    def k(table_hbm, idx_hbm, out_hbm, idx_v0, idx_v1, rows_v, pack_v,
          *sems):
        idx_vs = (idx_v0, idx_v1)
        gsem = sems[0:2]
        osem = sems[2:4]
        isem = sems[4:6]
        wid = lax.axis_index("s") * NC + lax.axis_index("c")
        bbase = wid * per_w
        fbase = bbase * S

        def stage_idx(q, slot):
            pltpu.async_copy(
                idx_hbm.at[pl.ds(fbase + q * (GB * S), GB * S)],
                idx_vs[slot], isem[slot])

        def wait_idx(slot):
            pltpu.make_async_copy(
                idx_hbm.at[pl.ds(0, GB * S)], idx_vs[slot],
                isem[slot]).wait()

        def fire(t, slot):
            g = (t // GB) & 1
            off = (t % GB) * S
            pltpu.async_copy(
                table_hbm.at[idx_vs[g].at[pl.ds(off, 128)]],
                rows_v.at[slot, pl.ds(0, 128)], gsem[slot])
            pltpu.async_copy(
                table_hbm.at[idx_vs[g].at[pl.ds(off + 128, S - 128)]],
                rows_v.at[slot, pl.ds(128, S - 128)], gsem[slot])

        def drain_gather(slot):
            pltpu.make_async_copy(
                table_hbm.at[pl.ds(0, S)], rows_v.at[slot],
                gsem[slot]).wait()

        def repack(slot):
            def rbody(rr, carry):
                for u in range(4):
                    r = rr * 4 + u
                    for c in range(4):
                        pack_v[slot, r, pl.ds(16 * c, 16)] = (
                            rows_v[slot, r, pl.ds(16 * c, 16)])
                return carry
            lax.fori_loop(0, S // 4, rbody, 0)

        def start_out(i, slot):
            pltpu.async_copy(pack_v.at[slot], out_hbm.at[bbase + i],
                             osem[slot])

        def wait_out(slot):
            pltpu.make_async_copy(pack_v.at[slot], out_hbm.at[0],
                                  osem[slot]).wait()

        def round_body(r, first=False, last=False):
            for t in range(2 * GB):
                i = r * (2 * GB) + t
                slot = t & 1
                g = (t // GB) & 1
                drain_gather(slot)
                if t % GB == GB - 1 and not last:
                    stage_idx(r * 2 + (t // GB) + 2, g)
                if not (first and t < 2):
                    wait_out(slot)
                repack(slot)
                start_out(i, slot)
                if t % GB == GB - 2 and not (last and t // GB == 1):
                    wait_idx(g ^ 1)
                if not (last and t >= 2 * GB - 2):
                    fire((t + 2) % (2 * GB), slot)

        pltpu.sync_copy(idx_hbm.at[pl.ds(fbase, GB * S)], idx_vs[0])
        stage_idx(1, 1)
        fire(0, 0)
        fire(1, 1)

        round_body(0, first=True)

        def loop_body(r, carry):
            round_body(r)
            return carry

        lax.fori_loop(1, n_groups // 2 - 1, loop_body, 0)

        round_body(n_groups // 2 - 1, last=True)
        wait_out(0)
        wait_out(1)

    return k


@jax.jit
def kernel(pos_ids, position_encoding):
    b, s = pos_ids.shape
    idx = pos_ids.reshape(b * s).astype(jnp.int32)
    table = jnp.pad(position_encoding.astype(jnp.float32),
                    ((0, 0), (0, DP - D)))
    return _gather_call(b)(table, idx)
